# baseline (speedup 1.0000x reference)
import functools

import jax
import jax.numpy as jnp
from jax import lax
from jax.experimental import pallas as pl
from jax.experimental.pallas import tpu as pltpu
from jax.experimental.pallas import tpu_sc as plsc

GRID = 300
DIM = 48
N = 262144
L = 16
B = 128
NW = 32
PPW = N // NW
NCHUNK = PPW // B
MODES = ((0, 1, 2), (0, 2, 1), (1, 2, 0))


def _sc_body(xs, ys, zs, pt0, pt1, pt2, lt0, lt1, lt2,
             o0, o1, o2,
             xv, yv, zv,
             i00, i01, i10, i11,
             wa, wb, wc, wd,
             li0, li1, lw0, lw1,
             g00, g01, g10, g11,
             tl0, tl1, tl2, ot, sem):
    wid = lax.axis_index("s") * 2 + lax.axis_index("c")
    base = wid * PPW
    coords = (xv, yv, zv)
    planes = (pt0, pt1, pt2)
    ltabs = (tl0, tl1, tl2)
    outs = (o0, o1, o2)

    pltpu.sync_copy(lt0, tl0)
    pltpu.sync_copy(lt1, tl1)
    pltpu.sync_copy(lt2, tl2)

    def chunk_body(k, _):
        p0 = base + k * B
        pltpu.sync_copy(xs.at[pl.ds(p0, B)], xv)
        pltpu.sync_copy(ys.at[pl.ds(p0, B)], yv)
        pltpu.sync_copy(zs.at[pl.ds(p0, B)], zv)

        for m in range(3):
            ub, hb, lb = (coords[c] for c in MODES[m])

            def idx_body(g, _, ub=ub, hb=hb, lb=lb):
                s = g * L
                u = ub[pl.ds(s, L)]
                h = hb[pl.ds(s, L)]
                v = lb[pl.ds(s, L)]
                ix = (u + 1.0) * 0.5 * (GRID - 1)
                iy = (h + 1.0) * 0.5 * (GRID - 1)
                iv = (v + 1.0) * 0.5 * (GRID - 1)
                x0 = jnp.minimum(ix.astype(jnp.int32), GRID - 2)
                y0 = jnp.minimum(iy.astype(jnp.int32), GRID - 2)
                v0 = jnp.minimum(iv.astype(jnp.int32), GRID - 2)
                wx = ix - x0.astype(jnp.float32)
                wy = iy - y0.astype(jnp.float32)
                wv = iv - v0.astype(jnp.float32)
                r00 = y0 * GRID + x0
                i00[pl.ds(s, L)] = r00
                i01[pl.ds(s, L)] = r00 + 1
                i10[pl.ds(s, L)] = r00 + GRID
                i11[pl.ds(s, L)] = r00 + GRID + 1
                ex = 1.0 - wx
                ey = 1.0 - wy
                wa[pl.ds(s, L)] = ex * ey
                wb[pl.ds(s, L)] = wx * ey
                wc[pl.ds(s, L)] = ex * wy
                wd[pl.ds(s, L)] = wx * wy
                li0[pl.ds(s, L)] = v0
                li1[pl.ds(s, L)] = v0 + 1
                lw0[pl.ds(s, L)] = 1.0 - wv
                lw1[pl.ds(s, L)] = wv
                return 0

            lax.fori_loop(0, B // L, idx_body, 0, unroll=False)

            h0 = pltpu.async_copy(planes[m].at[i00], g00, sem)
            h1 = pltpu.async_copy(planes[m].at[i01], g01, sem)
            h2 = pltpu.async_copy(planes[m].at[i10], g10, sem)
            h3 = pltpu.async_copy(planes[m].at[i11], g11, sem)
            h0.wait()
            h1.wait()
            h2.wait()
            h3.wait()

            tl = ltabs[m]

            def grp_body(g, _, tl=tl):
                s = g * L
                w00 = wa[pl.ds(s, L)]
                w01 = wb[pl.ds(s, L)]
                w10 = wc[pl.ds(s, L)]
                w11 = wd[pl.ds(s, L)]
                l0w = lw0[pl.ds(s, L)]
                l1w = lw1[pl.ds(s, L)]
                l0i = li0[pl.ds(s, L)]
                l1i = li1[pl.ds(s, L)]
                rows = lax.iota(jnp.int32, L) + s
                lanes = lax.iota(jnp.int32, L) + s

                def ch_body(cq, _):
                    for j in range(4):
                        c = cq * 4 + j
                        cs = jnp.full((L,), c, jnp.int32)
                        v00 = plsc.load_gather(g00, [rows, cs])
                        v01 = plsc.load_gather(g01, [rows, cs])
                        v10 = plsc.load_gather(g10, [rows, cs])
                        v11 = plsc.load_gather(g11, [rows, cs])
                        t0 = plsc.load_gather(tl, [l0i, cs])
                        t1 = plsc.load_gather(tl, [l1i, cs])
                        pcv = v00 * w00 + v01 * w01 + v10 * w10 + v11 * w11
                        lcv = t0 * l0w + t1 * l1w
                        plsc.store_scatter(ot, [cs, lanes], pcv * lcv)
                    return 0

                lax.fori_loop(0, DIM // 4, ch_body, 0, unroll=False)
                return 0

            lax.fori_loop(0, B // L, grp_body, 0, unroll=False)
            pltpu.sync_copy(ot, outs[m].at[:, pl.ds(p0, B)])
        return 0

    lax.fori_loop(0, NCHUNK, chunk_body, 0, unroll=False)


@functools.cache
def _build_sc_call():
  return functools.partial(
    pl.kernel,
    out_type=tuple(jax.ShapeDtypeStruct((DIM, N), jnp.float32) for _ in range(3)),
    mesh=plsc.VectorSubcoreMesh(core_axis_name="c", subcore_axis_name="s"),
    compiler_params=pltpu.CompilerParams(needs_layout_passes=False, use_tc_tiling_on_sc=False),
    scratch_types=[
        pltpu.VMEM((B,), jnp.float32),
        pltpu.VMEM((B,), jnp.float32),
        pltpu.VMEM((B,), jnp.float32),
        pltpu.VMEM((B,), jnp.int32),
        pltpu.VMEM((B,), jnp.int32),
        pltpu.VMEM((B,), jnp.int32),
        pltpu.VMEM((B,), jnp.int32),
        pltpu.VMEM((B,), jnp.float32),
        pltpu.VMEM((B,), jnp.float32),
        pltpu.VMEM((B,), jnp.float32),
        pltpu.VMEM((B,), jnp.float32),
        pltpu.VMEM((B,), jnp.int32),
        pltpu.VMEM((B,), jnp.int32),
        pltpu.VMEM((B,), jnp.float32),
        pltpu.VMEM((B,), jnp.float32),
        pltpu.VMEM((B, DIM), jnp.float32),
        pltpu.VMEM((B, DIM), jnp.float32),
        pltpu.VMEM((B, DIM), jnp.float32),
        pltpu.VMEM((B, DIM), jnp.float32),
        pltpu.VMEM((GRID, DIM), jnp.float32),
        pltpu.VMEM((GRID, DIM), jnp.float32),
        pltpu.VMEM((GRID, DIM), jnp.float32),
        pltpu.VMEM((DIM, B), jnp.float32),
        pltpu.SemaphoreType.DMA,
    ],
  )(_sc_body)


def kernel(xyz_normed, plane0, plane1, plane2, line0, line1, line2):
    xs = xyz_normed[:, 0]
    ys = xyz_normed[:, 1]
    zs = xyz_normed[:, 2]
    pts = [jnp.transpose(p, (1, 2, 0)).reshape(GRID * GRID, DIM)
           for p in (plane0, plane1, plane2)]
    lts = [jnp.transpose(l[:, :, 0], (1, 0)) for l in (line0, line1, line2)]
    return _build_sc_call()(xs, ys, zs, *pts, *lts)


# speedup vs baseline: 2.0116x; 2.0116x over previous
import functools

import jax
import jax.numpy as jnp
from jax import lax
from jax.experimental import pallas as pl
from jax.experimental.pallas import tpu as pltpu
from jax.experimental.pallas import tpu_sc as plsc

GRID = 300
DIM = 48
N = 262144
L = 16
B = 128
NW = 32
PPW = N // NW
NCHUNK = PPW // B
MODES = ((0, 1, 2), (0, 2, 1), (1, 2, 0))


def _sc_body(xs, ys, zs, pt0, pt1, pt2, lt0, lt1, lt2,
             o0, o1, o2,
             xv, yv, zv,
             ia0, ib0, ic0, id0, ia1, ib1, ic1, id1,
             wa0, wb0, wc0, wd0, wa1, wb1, wc1, wd1,
             li00, li10, lw00, lw10, li01, li11, lw01, lw11,
             ga0, gb0, gc0, gd0, ga1, gb1, gc1, gd1,
             tl0, tl1, tl2, ot, sem):
    wid = lax.axis_index("s") * 2 + lax.axis_index("c")
    base = wid * PPW
    coords = (xv, yv, zv)
    planes = (pt0, pt1, pt2)
    ltabs = (tl0, tl1, tl2)
    outs = (o0, o1, o2)
    idxb = ((ia0, ib0, ic0, id0), (ia1, ib1, ic1, id1))
    wgtb = ((wa0, wb0, wc0, wd0, li00, li10, lw00, lw10),
            (wa1, wb1, wc1, wd1, li01, li11, lw01, lw11))
    gb = ((ga0, gb0, gc0, gd0), (ga1, gb1, gc1, gd1))

    pltpu.sync_copy(lt0, tl0)
    pltpu.sync_copy(lt1, tl1)
    pltpu.sync_copy(lt2, tl2)

    def load_coords(k):
        p0 = base + k * B
        pltpu.sync_copy(xs.at[pl.ds(p0, B)], xv)
        pltpu.sync_copy(ys.at[pl.ds(p0, B)], yv)
        pltpu.sync_copy(zs.at[pl.ds(p0, B)], zv)

    def calc_idx(m, q):
        ub, hb, lb = (coords[c] for c in MODES[m])
        i00, i01, i10, i11 = idxb[q]
        wa, wb, wc, wd, li0, li1, lw0, lw1 = wgtb[q]

        def idx_body(g, _):
            s = g * L
            u = ub[pl.ds(s, L)]
            h = hb[pl.ds(s, L)]
            v = lb[pl.ds(s, L)]
            ix = (u + 1.0) * 0.5 * (GRID - 1)
            iy = (h + 1.0) * 0.5 * (GRID - 1)
            iv = (v + 1.0) * 0.5 * (GRID - 1)
            x0 = jnp.minimum(ix.astype(jnp.int32), GRID - 2)
            y0 = jnp.minimum(iy.astype(jnp.int32), GRID - 2)
            v0 = jnp.minimum(iv.astype(jnp.int32), GRID - 2)
            wx = ix - x0.astype(jnp.float32)
            wy = iy - y0.astype(jnp.float32)
            wv = iv - v0.astype(jnp.float32)
            r00 = y0 * GRID + x0
            i00[pl.ds(s, L)] = r00
            i01[pl.ds(s, L)] = r00 + 1
            i10[pl.ds(s, L)] = r00 + GRID
            i11[pl.ds(s, L)] = r00 + GRID + 1
            ex = 1.0 - wx
            ey = 1.0 - wy
            wa[pl.ds(s, L)] = ex * ey
            wb[pl.ds(s, L)] = wx * ey
            wc[pl.ds(s, L)] = ex * wy
            wd[pl.ds(s, L)] = wx * wy
            li0[pl.ds(s, L)] = v0
            li1[pl.ds(s, L)] = v0 + 1
            lw0[pl.ds(s, L)] = 1.0 - wv
            lw1[pl.ds(s, L)] = wv
            return 0

        lax.fori_loop(0, B // L, idx_body, 0, unroll=False)

    def fire_gathers(m, q):
        for i in range(4):
            pltpu.async_copy(planes[m].at[idxb[q][i]], gb[q][i], sem)

    def wait_gathers(m, q):
        dummy = planes[m].at[pl.ds(0, B)]
        for i in range(4):
            pltpu.make_async_copy(dummy, gb[q][i], sem).wait()

    def compute(m, k, p):
        g00, g01, g10, g11 = gb[p]
        wa, wb, wc, wd, li0, li1, lw0, lw1 = wgtb[p]
        tl = ltabs[m]
        p0 = base + k * B

        def pt_body(b2, _):
            for j in range(2):
                b = b2 * 2 + j
                bs = jnp.full((L,), b, jnp.int32)
                w00 = plsc.load_gather(wa, [bs])
                w01 = plsc.load_gather(wb, [bs])
                w10 = plsc.load_gather(wc, [bs])
                w11 = plsc.load_gather(wd, [bs])
                l0w = plsc.load_gather(lw0, [bs])
                l1w = plsc.load_gather(lw1, [bs])
                r0 = plsc.load_gather(li0, [bs]) * DIM
                r1 = plsc.load_gather(li1, [bs]) * DIM
                ci = lax.iota(jnp.int32, L)
                for cg in range(DIM // L):
                    sl = pl.ds(cg * L, L)
                    v00 = g00[b, sl]
                    v01 = g01[b, sl]
                    v10 = g10[b, sl]
                    v11 = g11[b, sl]
                    t0 = plsc.load_gather(tl, [r0 + (ci + cg * L)])
                    t1 = plsc.load_gather(tl, [r1 + (ci + cg * L)])
                    pcv = v00 * w00 + v01 * w01 + v10 * w10 + v11 * w11
                    lcv = t0 * l0w + t1 * l1w
                    ot[b, sl] = pcv * lcv
            return 0

        lax.fori_loop(0, B // 2, pt_body, 0, unroll=False)
        pltpu.sync_copy(ot, outs[m].at[pl.ds(p0, B)])

    load_coords(0)
    calc_idx(0, 0)
    fire_gathers(0, 0)

    def outer_body(k2, _):
        for kk in range(2):
            k = k2 * 2 + kk
            for m in range(3):
                s = kk * 3 + m
                p = s % 2
                q = 1 - p
                wait_gathers(m, p)
                if m == 2:
                    kn = jnp.minimum(k + 1, NCHUNK - 1)
                    load_coords(kn)
                mn = (m + 1) % 3
                calc_idx(mn, q)
                fire_gathers(mn, q)
                compute(m, k, p)
        return 0

    lax.fori_loop(0, NCHUNK // 2, outer_body, 0, unroll=False)
    wait_gathers(0, 0)


@functools.cache
def _build_sc_call():
  vf = functools.partial(pltpu.VMEM, (B,))
  return functools.partial(
    pl.kernel,
    out_type=tuple(jax.ShapeDtypeStruct((N, DIM), jnp.float32) for _ in range(3)),
    mesh=plsc.VectorSubcoreMesh(core_axis_name="c", subcore_axis_name="s"),
    compiler_params=pltpu.CompilerParams(needs_layout_passes=False,
                                         use_tc_tiling_on_sc=False),
    scratch_types=(
        [vf(jnp.float32) for _ in range(3)]
        + [vf(jnp.int32) for _ in range(8)]
        + [vf(jnp.float32) for _ in range(4)]
        + [vf(jnp.float32) for _ in range(4)]
        + [vf(jnp.int32), vf(jnp.int32), vf(jnp.float32), vf(jnp.float32)]
        + [vf(jnp.int32), vf(jnp.int32), vf(jnp.float32), vf(jnp.float32)]
        + [pltpu.VMEM((B, DIM), jnp.float32) for _ in range(8)]
        + [pltpu.VMEM((GRID * DIM,), jnp.float32) for _ in range(3)]
        + [pltpu.VMEM((B, DIM), jnp.float32)]
        + [pltpu.SemaphoreType.DMA]
    ),
  )(_sc_body)


def kernel(xyz_normed, plane0, plane1, plane2, line0, line1, line2):
    xs = xyz_normed[:, 0]
    ys = xyz_normed[:, 1]
    zs = xyz_normed[:, 2]
    pts = [jnp.transpose(p, (1, 2, 0)).reshape(GRID * GRID, DIM)
           for p in (plane0, plane1, plane2)]
    lts = [jnp.transpose(l[:, :, 0], (1, 0)).reshape(GRID * DIM)
           for l in (line0, line1, line2)]
    f0, f1, f2 = _build_sc_call()(xs, ys, zs, *pts, *lts)
    return (f0.T, f1.T, f2.T)


# speedup vs baseline: 2.8760x; 1.4297x over previous
import functools

import jax
import jax.numpy as jnp
from jax import lax
from jax.experimental import pallas as pl
from jax.experimental.pallas import tpu as pltpu
from jax.experimental.pallas import tpu_sc as plsc

GRID = 300
DIM = 48
N = 262144
L = 16
B = 128
NW = 32
PPW = N // NW
NCHUNK = PPW // B
MODES = ((0, 1, 2), (0, 2, 1), (1, 2, 0))


def _sc_body(xs, ys, zs, pt0, pt1, pt2, lt0, lt1, lt2,
             o0, o1, o2,
             xv, yv, zv,
             ia0, ib0, ic0, id0, ia1, ib1, ic1, id1,
             wx0, wy0, lw0_, li0_, wx1, wy1, lw1_, li1_,
             ga0, gb0, gc0, gd0, ga1, gb1, gc1, gd1,
             tl0, tl1, tl2, ot, sem):
    wid = lax.axis_index("s") * 2 + lax.axis_index("c")
    base = wid * PPW
    coords = (xv, yv, zv)
    planes = (pt0, pt1, pt2)
    ltabs = (tl0, tl1, tl2)
    outs = (o0, o1, o2)
    idxb = ((ia0, ib0, ic0, id0), (ia1, ib1, ic1, id1))
    wgtb = ((wx0, wy0, lw0_, li0_), (wx1, wy1, lw1_, li1_))
    gb = ((ga0, gb0, gc0, gd0), (ga1, gb1, gc1, gd1))

    pltpu.sync_copy(lt0, tl0)
    pltpu.sync_copy(lt1, tl1)
    pltpu.sync_copy(lt2, tl2)

    def load_coords(k):
        p0 = base + k * B
        pltpu.sync_copy(xs.at[pl.ds(p0, B)], xv)
        pltpu.sync_copy(ys.at[pl.ds(p0, B)], yv)
        pltpu.sync_copy(zs.at[pl.ds(p0, B)], zv)

    def calc_idx(m, q):
        ub, hb, lb = (coords[c] for c in MODES[m])
        i00, i01, i10, i11 = idxb[q]
        wxb, wyb, lwb, lib = wgtb[q]

        def idx_body(g, _):
            s = g * L
            u = ub[pl.ds(s, L)]
            h = hb[pl.ds(s, L)]
            v = lb[pl.ds(s, L)]
            ix = (u + 1.0) * 0.5 * (GRID - 1)
            iy = (h + 1.0) * 0.5 * (GRID - 1)
            iv = (v + 1.0) * 0.5 * (GRID - 1)
            x0 = jnp.minimum(ix.astype(jnp.int32), GRID - 2)
            y0 = jnp.minimum(iy.astype(jnp.int32), GRID - 2)
            v0 = jnp.minimum(iv.astype(jnp.int32), GRID - 2)
            wx = ix - x0.astype(jnp.float32)
            wy = iy - y0.astype(jnp.float32)
            wv = iv - v0.astype(jnp.float32)
            r00 = y0 * GRID + x0
            i00[pl.ds(s, L)] = r00
            i01[pl.ds(s, L)] = r00 + 1
            i10[pl.ds(s, L)] = r00 + GRID
            i11[pl.ds(s, L)] = r00 + GRID + 1
            wxb[pl.ds(s, L)] = wx
            wyb[pl.ds(s, L)] = wy
            lwb[pl.ds(s, L)] = wv
            lib[pl.ds(s, L)] = v0 * DIM
            return 0

        lax.fori_loop(0, B // L, idx_body, 0, unroll=False)

    def fire_gathers(m, q):
        for i in range(4):
            pltpu.async_copy(planes[m].at[idxb[q][i]], gb[q][i], sem)

    def wait_gathers(m, q):
        dummy = planes[m].at[pl.ds(0, B)]
        for i in range(4):
            pltpu.make_async_copy(dummy, gb[q][i], sem).wait()

    def compute(m, k, p):
        g00, g01, g10, g11 = gb[p]
        wxb, wyb, lwb, lib = wgtb[p]
        tl = ltabs[m]
        p0 = base + k * B
        ci = lax.iota(jnp.int32, L)

        @plsc.parallel_loop(0, B, 1, unroll=4)
        def pt_body(b):
            bs = jnp.full((L,), b, jnp.int32)
            wx = plsc.load_gather(wxb, [bs])
            wy = plsc.load_gather(wyb, [bs])
            lw = plsc.load_gather(lwb, [bs])
            r0 = plsc.load_gather(lib, [bs])
            r1 = r0 + DIM
            for cg in range(DIM // L):
                sl = pl.ds(cg * L, L)
                v00 = g00[b, sl]
                v01 = g01[b, sl]
                v10 = g10[b, sl]
                v11 = g11[b, sl]
                t0 = plsc.load_gather(tl, [r0 + (ci + cg * L)])
                t1 = plsc.load_gather(tl, [r1 + (ci + cg * L)])
                a = v00 + wx * (v01 - v00)
                bb = v10 + wx * (v11 - v10)
                pcv = a + wy * (bb - a)
                lcv = t0 + lw * (t1 - t0)
                ot[b, sl] = pcv * lcv

        pltpu.sync_copy(ot, outs[m].at[pl.ds(p0, B)])

    load_coords(0)
    calc_idx(0, 0)
    fire_gathers(0, 0)

    def outer_body(k2, _):
        for kk in range(2):
            k = k2 * 2 + kk
            for m in range(3):
                s = kk * 3 + m
                p = s % 2
                q = 1 - p
                wait_gathers(m, p)
                if m == 2:
                    kn = jnp.minimum(k + 1, NCHUNK - 1)
                    load_coords(kn)
                mn = (m + 1) % 3
                calc_idx(mn, q)
                fire_gathers(mn, q)
                compute(m, k, p)
        return 0

    lax.fori_loop(0, NCHUNK // 2, outer_body, 0, unroll=False)
    wait_gathers(0, 0)


@functools.cache
def _build_sc_call():
  vf = functools.partial(pltpu.VMEM, (B,))
  return functools.partial(
    pl.kernel,
    out_type=tuple(jax.ShapeDtypeStruct((N, DIM), jnp.float32) for _ in range(3)),
    mesh=plsc.VectorSubcoreMesh(core_axis_name="c", subcore_axis_name="s"),
    compiler_params=pltpu.CompilerParams(needs_layout_passes=False,
                                         use_tc_tiling_on_sc=False),
    scratch_types=(
        [vf(jnp.float32) for _ in range(3)]
        + [vf(jnp.int32) for _ in range(8)]
        + [vf(jnp.float32), vf(jnp.float32), vf(jnp.float32), vf(jnp.int32)]
        + [vf(jnp.float32), vf(jnp.float32), vf(jnp.float32), vf(jnp.int32)]
        + [pltpu.VMEM((B, DIM), jnp.float32) for _ in range(8)]
        + [pltpu.VMEM((GRID * DIM,), jnp.float32) for _ in range(3)]
        + [pltpu.VMEM((B, DIM), jnp.float32)]
        + [pltpu.SemaphoreType.DMA]
    ),
  )(_sc_body)


def kernel(xyz_normed, plane0, plane1, plane2, line0, line1, line2):
    xs = xyz_normed[:, 0]
    ys = xyz_normed[:, 1]
    zs = xyz_normed[:, 2]
    pts = [jnp.transpose(p, (1, 2, 0)).reshape(GRID * GRID, DIM)
           for p in (plane0, plane1, plane2)]
    lts = [jnp.transpose(l[:, :, 0], (1, 0)).reshape(GRID * DIM)
           for l in (line0, line1, line2)]
    f0, f1, f2 = _build_sc_call()(xs, ys, zs, *pts, *lts)
    return (f0.T, f1.T, f2.T)
